# baseline (speedup 1.0000x reference)
import jax
import jax.numpy as jnp
from jax.experimental import pallas as pl
from jax.experimental.pallas import tpu as pltpu

_DIM = 16
_SLOTS = 8
_ROWS = 12800
_RBLK = 1280
_PAD_COORD = 100.0


def _distnet_kernel(x_ref, flat_ref, beta_ref, out_ref, w_ref, xx_ref):
    i = pl.program_id(0)
    n = pl.num_programs(0)
    q = x_ref.shape[0]

    @pl.when(i == 0)
    def _init():
        x = x_ref[...]
        xt = x.T
        xx_ref[...] = jnp.sum(xt * xt, axis=0, keepdims=True)
        xtb = (-2.0 * xt).astype(jnp.bfloat16)
        w_ref[...] = jnp.zeros(w_ref.shape, jnp.bfloat16)
        for j in range(_SLOTS):
            w_ref[_DIM * j:_DIM * (j + 1), q * j:q * (j + 1)] = xtb
            w_ref[128 + j:129 + j, q * j:q * (j + 1)] = jnp.ones(
                (1, q), jnp.bfloat16)

    flat = flat_ref[...]
    flatb = flat.astype(jnp.bfloat16)
    sq = (flat * flat).astype(jnp.bfloat16)
    k_idx = jax.lax.broadcasted_iota(jnp.int32, (_SLOTS * _DIM, _SLOTS), 0)
    j_idx = jax.lax.broadcasted_iota(jnp.int32, (_SLOTS * _DIM, _SLOTS), 1)
    seg = (k_idx // _DIM == j_idx).astype(jnp.bfloat16)
    pp8 = jax.lax.dot_general(
        sq, seg, (((1,), (0,)), ((), ())),
        preferred_element_type=jnp.float32)
    lhs = jnp.concatenate([flatb, pp8.astype(jnp.bfloat16)], axis=1)

    m = None
    for j in range(_SLOTS):
        w = w_ref[:, q * j:q * (j + 1)]
        part = jax.lax.dot_general(
            lhs, w, (((1,), (0,)), ((), ())),
            preferred_element_type=jnp.float32)
        mj = jnp.min(part, axis=0, keepdims=True)
        m = mj if m is None else jnp.minimum(m, mj)

    @pl.when(i == 0)
    def _first():
        out_ref[...] = m

    @pl.when(i > 0)
    def _acc():
        out_ref[...] = jnp.minimum(out_ref[...], m)

    @pl.when(i == n - 1)
    def _fin():
        d = jnp.maximum(out_ref[...] + xx_ref[...], 0.0)
        b = jax.nn.softplus(beta_ref[0, 0])
        alpha = -b * 6.9077542789816375
        out_ref[...] = jax.nn.sigmoid((d + alpha) / b)


def kernel(x, points, beta):
    q, dim = x.shape
    n_pts = points.shape[0]
    lanes = _SLOTS * _DIM
    assert dim == _DIM and n_pts * dim == 12500 * lanes
    flat = points.reshape(12500, lanes)
    flat = jnp.pad(flat, ((0, _ROWS - 12500), (0, 0)),
                   constant_values=_PAD_COORD)
    beta2d = beta.reshape(1, 1)
    n_steps = _ROWS // _RBLK
    out = pl.pallas_call(
        _distnet_kernel,
        grid=(n_steps,),
        in_specs=[
            pl.BlockSpec((q, dim), lambda i: (0, 0)),
            pl.BlockSpec((_RBLK, lanes), lambda i: (i, 0)),
            pl.BlockSpec((1, 1), lambda i: (0, 0)),
        ],
        out_specs=pl.BlockSpec((1, q), lambda i: (0, 0)),
        out_shape=jax.ShapeDtypeStruct((1, q), jnp.float32),
        scratch_shapes=[
            pltpu.VMEM((lanes + _SLOTS, _SLOTS * q), jnp.bfloat16),
            pltpu.VMEM((1, q), jnp.float32),
        ],
    )(x, flat, beta2d)
    return out.reshape(q)


# speedup vs baseline: 1.1461x; 1.1461x over previous
import jax
import jax.numpy as jnp
from jax.experimental import pallas as pl
from jax.experimental.pallas import tpu as pltpu

_DIM = 16
_SLOTS = 8
_ROWS = 12800
_RBLK = 1280
_PAD_COORD = 100.0


def _distnet_kernel(x_ref, flat_ref, beta_ref, out_ref, w_ref, xx_ref):
    i = pl.program_id(0)
    n = pl.num_programs(0)
    q = x_ref.shape[0]

    @pl.when(i == 0)
    def _init():
        x = x_ref[...]
        xt = x.T
        xx_ref[...] = jnp.sum(xt * xt, axis=0, keepdims=True)
        xtb = (-2.0 * xt).astype(jnp.bfloat16)
        w_ref[...] = jnp.zeros(w_ref.shape, jnp.bfloat16)
        for j in range(_SLOTS):
            w_ref[_DIM * j:_DIM * (j + 1), q * j:q * (j + 1)] = xtb
            w_ref[128 + j:129 + j, q * j:q * (j + 1)] = jnp.ones(
                (1, q), jnp.bfloat16)

    flatb = flat_ref[...]
    sq = flatb * flatb
    k_idx = jax.lax.broadcasted_iota(jnp.int32, (_SLOTS * _DIM, _SLOTS), 0)
    j_idx = jax.lax.broadcasted_iota(jnp.int32, (_SLOTS * _DIM, _SLOTS), 1)
    seg = (k_idx // _DIM == j_idx).astype(jnp.bfloat16)
    pp8 = jax.lax.dot_general(
        sq, seg, (((1,), (0,)), ((), ())),
        preferred_element_type=jnp.float32)
    lhs = jnp.concatenate([flatb, pp8.astype(jnp.bfloat16)], axis=1)

    m = None
    for j in range(_SLOTS):
        w = w_ref[:, q * j:q * (j + 1)]
        part = jax.lax.dot_general(
            lhs, w, (((1,), (0,)), ((), ())),
            preferred_element_type=jnp.float32)
        mj = jnp.min(part, axis=0, keepdims=True)
        m = mj if m is None else jnp.minimum(m, mj)

    @pl.when(i == 0)
    def _first():
        out_ref[...] = m

    @pl.when(i > 0)
    def _acc():
        out_ref[...] = jnp.minimum(out_ref[...], m)

    @pl.when(i == n - 1)
    def _fin():
        d = jnp.maximum(out_ref[...] + xx_ref[...], 0.0)
        b = jax.nn.softplus(beta_ref[0, 0])
        alpha = -b * 6.9077542789816375
        out_ref[...] = jax.nn.sigmoid((d + alpha) / b)


def kernel(x, points, beta):
    q, dim = x.shape
    n_pts = points.shape[0]
    lanes = _SLOTS * _DIM
    assert dim == _DIM and n_pts * dim == 12500 * lanes
    flat = points.astype(jnp.bfloat16).reshape(12500, lanes)
    flat = jnp.pad(flat, ((0, _ROWS - 12500), (0, 0)),
                   constant_values=_PAD_COORD)
    beta2d = beta.reshape(1, 1)
    n_steps = _ROWS // _RBLK
    out = pl.pallas_call(
        _distnet_kernel,
        grid=(n_steps,),
        in_specs=[
            pl.BlockSpec((q, dim), lambda i: (0, 0)),
            pl.BlockSpec((_RBLK, lanes), lambda i: (i, 0)),
            pl.BlockSpec((1, 1), lambda i: (0, 0)),
        ],
        out_specs=pl.BlockSpec((1, q), lambda i: (0, 0)),
        out_shape=jax.ShapeDtypeStruct((1, q), jnp.float32),
        scratch_shapes=[
            pltpu.VMEM((lanes + _SLOTS, _SLOTS * q), jnp.bfloat16),
            pltpu.VMEM((1, q), jnp.float32),
        ],
    )(x, flat, beta2d)
    return out.reshape(q)


# speedup vs baseline: 1.3280x; 1.1587x over previous
import jax
import jax.numpy as jnp
from jax.experimental import pallas as pl
from jax.experimental.pallas import tpu as pltpu

_BLOCK = 10000
_SUB = 2000


def _distnet_kernel(x_ref, pts_hbm, beta_ref, out_ref, buf_ref, sem):
    i = pl.program_id(0)
    n = pl.num_programs(0)

    slot = jax.lax.rem(i, 2)
    @pl.when(i == 0)
    def _start_first():
        pltpu.make_async_copy(
            pts_hbm.at[pl.ds(0, _BLOCK), :], buf_ref.at[0], sem.at[0]
        ).start()

    @pl.when(i + 1 < n)
    def _prefetch_next():
        pltpu.make_async_copy(
            pts_hbm.at[pl.ds((i + 1) * _BLOCK, _BLOCK), :],
            buf_ref.at[1 - slot], sem.at[1 - slot],
        ).start()

    pltpu.make_async_copy(
        pts_hbm.at[pl.ds(i * _BLOCK, _BLOCK), :], buf_ref.at[slot],
        sem.at[slot],
    ).wait()

    x = x_ref[...]
    pts = buf_ref[slot]
    pts_t = pts.T
    pp = jnp.sum(pts_t * pts_t, axis=0, keepdims=True)
    lhs = jnp.concatenate(
        [jnp.ones((x.shape[0], 1), jnp.bfloat16), x.astype(jnp.bfloat16)],
        axis=1)
    rhs = jnp.concatenate([pp, -2.0 * pts_t], axis=0).astype(jnp.bfloat16)
    mblk = None
    for s in range(0, _BLOCK, _SUB):
        partial = jax.lax.dot_general(
            lhs, rhs[:, s:s + _SUB], (((1,), (0,)), ((), ())),
            preferred_element_type=jnp.float32)
        m = jnp.min(partial, axis=1, keepdims=True)
        mblk = m if mblk is None else jnp.minimum(mblk, m)

    @pl.when(i == 0)
    def _first():
        out_ref[...] = mblk

    @pl.when(i > 0)
    def _acc():
        out_ref[...] = jnp.minimum(out_ref[...], mblk)

    @pl.when(i == n - 1)
    def _fin():
        xx = jnp.sum(x * x, axis=1, keepdims=True)
        d = jnp.maximum(out_ref[...] + xx, 0.0)
        b = jax.nn.softplus(beta_ref[0, 0])
        alpha = -b * 6.9077542789816375
        out_ref[...] = jax.nn.sigmoid((d + alpha) / b)


def kernel(x, points, beta):
    q, dim = x.shape
    n_pts = points.shape[0]
    assert n_pts % _BLOCK == 0, n_pts
    n_steps = n_pts // _BLOCK
    beta2d = beta.reshape(1, 1)
    out = pl.pallas_call(
        _distnet_kernel,
        grid=(n_steps,),
        in_specs=[
            pl.BlockSpec((q, dim), lambda i: (0, 0)),
            pl.BlockSpec(memory_space=pltpu.MemorySpace.HBM),
            pl.BlockSpec((1, 1), lambda i: (0, 0)),
        ],
        out_specs=pl.BlockSpec((q, 1), lambda i: (0, 0)),
        out_shape=jax.ShapeDtypeStruct((q, 1), jnp.float32),
        scratch_shapes=[
            pltpu.VMEM((2, _BLOCK, dim), jnp.float32),
            pltpu.SemaphoreType.DMA((2,)),
        ],
    )(x, points, beta2d)
    return out.reshape(q)


# speedup vs baseline: 1.7869x; 1.3455x over previous
import jax
import jax.numpy as jnp
from jax.experimental import pallas as pl

_NPAD = 102400
_BLOCK = 10240
_SUB = 2048
_PAD_COORD = 100.0


def _distnet_kernel(x_ref, ptst_ref, beta_ref, out_ref):
    i = pl.program_id(0)
    n = pl.num_programs(0)
    x = x_ref[...]
    pts_t = ptst_ref[...]
    pp = jnp.sum(pts_t * pts_t, axis=0, keepdims=True)
    lhs = jnp.concatenate(
        [jnp.ones((x.shape[0], 1), jnp.bfloat16), x.astype(jnp.bfloat16)],
        axis=1)
    rhs = jnp.concatenate([pp, -2.0 * pts_t], axis=0).astype(jnp.bfloat16)
    mblk = None
    for s in range(0, _BLOCK, _SUB):
        partial = jax.lax.dot_general(
            lhs, rhs[:, s:s + _SUB], (((1,), (0,)), ((), ())),
            preferred_element_type=jnp.float32)
        m = jnp.min(partial, axis=1, keepdims=True)
        mblk = m if mblk is None else jnp.minimum(mblk, m)

    @pl.when(i == 0)
    def _first():
        out_ref[...] = mblk

    @pl.when(i > 0)
    def _acc():
        out_ref[...] = jnp.minimum(out_ref[...], mblk)

    @pl.when(i == n - 1)
    def _fin():
        xx = jnp.sum(x * x, axis=1, keepdims=True)
        d = jnp.maximum(out_ref[...] + xx, 0.0)
        b = jax.nn.softplus(beta_ref[0, 0])
        alpha = -b * 6.9077542789816375
        out_ref[...] = jax.nn.sigmoid((d + alpha) / b)


def kernel(x, points, beta):
    q, dim = x.shape
    n_pts = points.shape[0]
    pts_t = jnp.pad(points.T, ((0, 0), (0, _NPAD - n_pts)),
                    constant_values=_PAD_COORD)
    beta2d = beta.reshape(1, 1)
    n_steps = _NPAD // _BLOCK
    out = pl.pallas_call(
        _distnet_kernel,
        grid=(n_steps,),
        in_specs=[
            pl.BlockSpec((q, dim), lambda i: (0, 0)),
            pl.BlockSpec((dim, _BLOCK), lambda i: (0, i)),
            pl.BlockSpec((1, 1), lambda i: (0, 0)),
        ],
        out_specs=pl.BlockSpec((q, 1), lambda i: (0, 0)),
        out_shape=jax.ShapeDtypeStruct((q, 1), jnp.float32),
    )(x, pts_t, beta2d)
    return out.reshape(q)


# speedup vs baseline: 1.7945x; 1.0043x over previous
import jax
import jax.numpy as jnp
from jax.experimental import pallas as pl

_NPAD = 102400
_BLOCK = 20480
_SUB = 1024
_PAD_COORD = 100.0


def _distnet_kernel(x_ref, ptst_ref, beta_ref, out_ref):
    i = pl.program_id(0)
    n = pl.num_programs(0)
    x = x_ref[...]
    pts_t = ptst_ref[...]
    pp = jnp.sum(pts_t * pts_t, axis=0, keepdims=True)
    lhs = jnp.concatenate(
        [jnp.ones((x.shape[0], 1), jnp.bfloat16), x.astype(jnp.bfloat16)],
        axis=1)
    rhs = jnp.concatenate([pp, -2.0 * pts_t], axis=0).astype(jnp.bfloat16)
    mblk = None
    for s in range(0, _BLOCK, _SUB):
        partial = jax.lax.dot_general(
            lhs, rhs[:, s:s + _SUB], (((1,), (0,)), ((), ())),
            preferred_element_type=jnp.float32)
        m = jnp.min(partial, axis=1, keepdims=True)
        mblk = m if mblk is None else jnp.minimum(mblk, m)

    @pl.when(i == 0)
    def _first():
        out_ref[...] = mblk

    @pl.when(i > 0)
    def _acc():
        out_ref[...] = jnp.minimum(out_ref[...], mblk)

    @pl.when(i == n - 1)
    def _fin():
        xx = jnp.sum(x * x, axis=1, keepdims=True)
        d = jnp.maximum(out_ref[...] + xx, 0.0)
        b = jax.nn.softplus(beta_ref[0, 0])
        alpha = -b * 6.9077542789816375
        out_ref[...] = jax.nn.sigmoid((d + alpha) / b)


def kernel(x, points, beta):
    q, dim = x.shape
    n_pts = points.shape[0]
    pts_t = jnp.pad(points.T, ((0, 0), (0, _NPAD - n_pts)),
                    constant_values=_PAD_COORD)
    beta2d = beta.reshape(1, 1)
    n_steps = _NPAD // _BLOCK
    out = pl.pallas_call(
        _distnet_kernel,
        grid=(n_steps,),
        in_specs=[
            pl.BlockSpec((q, dim), lambda i: (0, 0)),
            pl.BlockSpec((dim, _BLOCK), lambda i: (0, i)),
            pl.BlockSpec((1, 1), lambda i: (0, 0)),
        ],
        out_specs=pl.BlockSpec((q, 1), lambda i: (0, 0)),
        out_shape=jax.ShapeDtypeStruct((q, 1), jnp.float32),
    )(x, pts_t, beta2d)
    return out.reshape(q)
